# baseline (speedup 1.0000x reference)
import numpy as np
import jax
import jax.numpy as jnp
from jax.experimental import pallas as pl
from jax.experimental.pallas import tpu as pltpu

R = 64
C = 100000
C_BLK = 32768
NBLK = (C + C_BLK - 1) // C_BLK
W = 1024
CHUNKS = C_BLK // W
RG = R // 8
TAIL = C - (NBLK - 1) * C_BLK
TAIL_FULL = TAIL // W
TAIL_REM = TAIL - TAIL_FULL * W

_U32 = jnp.uint32
_KS = (np.uint32(0), np.uint32(42), np.uint32(42 ^ 0x1BD11BDA))
_TINY = np.float32(np.finfo(np.float32).tiny)
_IMAX = np.int32(np.iinfo(np.int32).max)


def _rotl(v, d):
    return (v << _U32(d)) | jax.lax.shift_right_logical(v, _U32(32 - d))


def _threefry_bits(a):
    x0 = a
    x1 = _rotl(a, 13) ^ a

    def rounds(x0, x1, rots):
        for r in rots:
            x0 = x0 + x1
            x1 = _rotl(x1, r) ^ x0
        return x0, x1

    x0, x1 = rounds(x0, x1, (15, 26, 6))
    x0 = x0 + _KS[1]
    x1 = x1 + np.uint32(_KS[2] + np.uint32(1))
    x0, x1 = rounds(x0, x1, (17, 29, 16, 24))
    x0 = x0 + _KS[2]
    x1 = x1 + np.uint32(2)
    x0, x1 = rounds(x0, x1, (13, 15, 26, 6))
    x1 = x1 + np.uint32(_KS[1] + np.uint32(3))
    x0, x1 = rounds(x0, x1, (17, 29, 16, 24))
    x0 = x0 + _KS[1]
    x1 = x1 + np.uint32(_KS[2] + np.uint32(4))
    x0, x1 = rounds(x0, x1, (13, 15, 26, 6))
    x0 = x0 + _KS[2]
    x1 = x1 + np.uint32(5)
    return x0 ^ x1


def _score(x_ref, base42, rg, ck, t):
    logits = x_ref[pl.ds(rg * 8, 8), pl.ds(ck * W, W)]
    a = base42 + (t.astype(_U32) * _U32(W) + _U32(np.uint32(rg * 8 * C)))
    bits = _threefry_bits(a)
    m = jax.lax.shift_right_logical(bits, _U32(9))
    f = jax.lax.convert_element_type(m, jnp.float32) * np.float32(2.0 ** -23)
    u = jnp.maximum(_TINY, f)
    nl2 = jnp.log(-jnp.log(u))
    return logits - nl2


def _sample_kernel(x_ref, o_ref, acc_val, acc_t):
    j = pl.program_id(0)

    @pl.when(j == 0)
    def _():
        acc_val[...] = jnp.full((R, W), -jnp.inf, jnp.float32)
        acc_t[...] = jnp.zeros((R, W), jnp.int32)

    lane = jax.lax.broadcasted_iota(jnp.int32, (8, W), 1)
    row_c = jax.lax.broadcasted_iota(jnp.int32, (8, W), 0) * np.int32(C)
    base42 = jax.lax.bitcast_convert_type(lane + row_c, _U32) + _U32(42)

    @pl.when(j != NBLK - 1)
    def _():
        for rg in range(RG):
            av = acc_val[pl.ds(rg * 8, 8), :]
            at = acc_t[pl.ds(rg * 8, 8), :]
            for ck in range(CHUNKS):
                t = j * np.int32(CHUNKS) + np.int32(ck)
                score = _score(x_ref, base42, rg, ck, t)
                take = score > av
                av = jnp.where(take, score, av)
                at = jnp.where(take, t, at)
            acc_val[pl.ds(rg * 8, 8), :] = av
            acc_t[pl.ds(rg * 8, 8), :] = at

    @pl.when(j == NBLK - 1)
    def _():
        for rg in range(RG):
            av = acc_val[pl.ds(rg * 8, 8), :]
            at = acc_t[pl.ds(rg * 8, 8), :]
            for ck in range(TAIL_FULL + (1 if TAIL_REM else 0)):
                t = np.int32((NBLK - 1) * CHUNKS + ck)
                score = _score(x_ref, base42, rg, ck, jnp.int32(t))
                take = score > av
                if ck >= TAIL_FULL:
                    take = take & (lane < np.int32(TAIL_REM))
                av = jnp.where(take, score, av)
                at = jnp.where(take, t, at)
            col = at * np.int32(W) + lane
            rmax = jnp.max(av, axis=1, keepdims=True)
            cand = jnp.where(av == rmax, col, _IMAX)
            o_ref[pl.ds(rg * 8, 8), :] = jnp.min(cand, axis=1, keepdims=True)


def kernel(logits):
    out = pl.pallas_call(
        _sample_kernel,
        grid=(NBLK,),
        in_specs=[pl.BlockSpec((R, C_BLK), lambda j: (0, j))],
        out_specs=pl.BlockSpec((R, 1), lambda j: (0, 0)),
        out_shape=jax.ShapeDtypeStruct((R, 1), jnp.int32),
        scratch_shapes=[
            pltpu.VMEM((R, W), jnp.float32),
            pltpu.VMEM((R, W), jnp.int32),
        ],
        compiler_params=pltpu.CompilerParams(
            dimension_semantics=("arbitrary",),
        ),
    )(logits)
    return out.reshape(R)


# speedup vs baseline: 1.5551x; 1.5551x over previous
import numpy as np
import jax
import jax.numpy as jnp
from jax.experimental import pallas as pl
from jax.experimental.pallas import tpu as pltpu

R = 64
C = 100000
C_BLK = 16384
NBLK = (C + C_BLK - 1) // C_BLK
W = 1024
CHUNKS = C_BLK // W
RG = R // 8
TAIL = C - (NBLK - 1) * C_BLK
TAIL_FULL = TAIL // W
TAIL_REM = TAIL - TAIL_FULL * W

_U32 = jnp.uint32
_KS = (np.uint32(0), np.uint32(42), np.uint32(42 ^ 0x1BD11BDA))
_TINY = np.float32(np.finfo(np.float32).tiny)
_IMAX = np.int32(np.iinfo(np.int32).max)


def _rotl(v, d):
    return (v << _U32(d)) | jax.lax.shift_right_logical(v, _U32(32 - d))


def _threefry_bits(a):
    x0 = a
    x1 = _rotl(a, 13) ^ a

    def rounds(x0, x1, rots):
        for r in rots:
            x0 = x0 + x1
            x1 = _rotl(x1, r) ^ x0
        return x0, x1

    x0, x1 = rounds(x0, x1, (15, 26, 6))
    x0 = x0 + _KS[1]
    x1 = x1 + np.uint32(_KS[2] + np.uint32(1))
    x0, x1 = rounds(x0, x1, (17, 29, 16, 24))
    x0 = x0 + _KS[2]
    x1 = x1 + np.uint32(2)
    x0, x1 = rounds(x0, x1, (13, 15, 26, 6))
    x1 = x1 + np.uint32(_KS[1] + np.uint32(3))
    x0, x1 = rounds(x0, x1, (17, 29, 16, 24))
    x0 = x0 + _KS[1]
    x1 = x1 + np.uint32(_KS[2] + np.uint32(4))
    x0, x1 = rounds(x0, x1, (13, 15, 26, 6))
    x0 = x0 + _KS[2]
    x1 = x1 + np.uint32(5)
    return x0 ^ x1


def _score(x_ref, base42, rg, ck, t):
    logits = x_ref[pl.ds(rg * 8, 8), pl.ds(ck * W, W)]
    a = base42 + (t.astype(_U32) * _U32(W) + _U32(np.uint32(rg * 8 * C)))
    bits = _threefry_bits(a)
    m = jax.lax.shift_right_logical(bits, _U32(9))
    f = jax.lax.convert_element_type(m, jnp.float32) * np.float32(2.0 ** -23)
    u = jnp.maximum(_TINY, f)
    t = jnp.log2(u) * np.float32(-0.6931471805599453)
    nl2 = jnp.log2(t) * np.float32(0.6931471805599453)
    return logits - nl2


def _sample_kernel(x_ref, o_ref, acc_val, acc_t):
    j = pl.program_id(0)

    @pl.when(j == 0)
    def _():
        acc_val[...] = jnp.full((R, W), -jnp.inf, jnp.float32)
        acc_t[...] = jnp.zeros((R, W), jnp.int32)

    lane = jax.lax.broadcasted_iota(jnp.int32, (8, W), 1)
    row_c = jax.lax.broadcasted_iota(jnp.int32, (8, W), 0) * np.int32(C)
    base42 = jax.lax.bitcast_convert_type(lane + row_c, _U32) + _U32(42)

    @pl.when(j != NBLK - 1)
    def _():
        for rg in range(RG):
            av = acc_val[pl.ds(rg * 8, 8), :]
            at = acc_t[pl.ds(rg * 8, 8), :]
            for ck in range(CHUNKS):
                t = j * np.int32(CHUNKS) + np.int32(ck)
                score = _score(x_ref, base42, rg, ck, t)
                take = score > av
                av = jnp.where(take, score, av)
                at = jnp.where(take, t, at)
            acc_val[pl.ds(rg * 8, 8), :] = av
            acc_t[pl.ds(rg * 8, 8), :] = at

    @pl.when(j == NBLK - 1)
    def _():
        for rg in range(RG):
            av = acc_val[pl.ds(rg * 8, 8), :]
            at = acc_t[pl.ds(rg * 8, 8), :]
            for ck in range(TAIL_FULL + (1 if TAIL_REM else 0)):
                t = np.int32((NBLK - 1) * CHUNKS + ck)
                score = _score(x_ref, base42, rg, ck, jnp.int32(t))
                take = score > av
                if ck >= TAIL_FULL:
                    take = take & (lane < np.int32(TAIL_REM))
                av = jnp.where(take, score, av)
                at = jnp.where(take, t, at)
            col = at * np.int32(W) + lane
            rmax = jnp.max(av, axis=1, keepdims=True)
            cand = jnp.where(av == rmax, col, _IMAX)
            o_ref[pl.ds(rg * 8, 8), :] = jnp.min(cand, axis=1, keepdims=True)


def kernel(logits):
    out = pl.pallas_call(
        _sample_kernel,
        grid=(NBLK,),
        in_specs=[pl.BlockSpec((R, C_BLK), lambda j: (0, j))],
        out_specs=pl.BlockSpec((R, 1), lambda j: (0, 0)),
        out_shape=jax.ShapeDtypeStruct((R, 1), jnp.int32),
        scratch_shapes=[
            pltpu.VMEM((R, W), jnp.float32),
            pltpu.VMEM((R, W), jnp.int32),
        ],
        compiler_params=pltpu.CompilerParams(
            dimension_semantics=("arbitrary",),
        ),
    )(logits)
    return out.reshape(R)


# speedup vs baseline: 1.5819x; 1.0172x over previous
import numpy as np
import jax
import jax.numpy as jnp
from jax.experimental import pallas as pl
from jax.experimental.pallas import tpu as pltpu

R = 64
C = 100000
C_BLK = 24576
NBLK = (C + C_BLK - 1) // C_BLK
W = 1024
CHUNKS = C_BLK // W
RG = R // 8
TAIL = C - (NBLK - 1) * C_BLK
TAIL_FULL = TAIL // W
TAIL_REM = TAIL - TAIL_FULL * W

_U32 = jnp.uint32
_KS = (np.uint32(0), np.uint32(42), np.uint32(42 ^ 0x1BD11BDA))
_TINY = np.float32(np.finfo(np.float32).tiny)
_IMAX = np.int32(np.iinfo(np.int32).max)


def _rotl(v, d):
    return (v << _U32(d)) | jax.lax.shift_right_logical(v, _U32(32 - d))


def _threefry_bits(a):
    x0 = a
    x1 = _rotl(a, 13) ^ a

    def rounds(x0, x1, rots):
        for r in rots:
            x0 = x0 + x1
            x1 = _rotl(x1, r) ^ x0
        return x0, x1

    x0, x1 = rounds(x0, x1, (15, 26, 6))
    x0 = x0 + _KS[1]
    x1 = x1 + np.uint32(_KS[2] + np.uint32(1))
    x0, x1 = rounds(x0, x1, (17, 29, 16, 24))
    x0 = x0 + _KS[2]
    x1 = x1 + np.uint32(2)
    x0, x1 = rounds(x0, x1, (13, 15, 26, 6))
    x1 = x1 + np.uint32(_KS[1] + np.uint32(3))
    x0, x1 = rounds(x0, x1, (17, 29, 16, 24))
    x0 = x0 + _KS[1]
    x1 = x1 + np.uint32(_KS[2] + np.uint32(4))
    x0, x1 = rounds(x0, x1, (13, 15, 26, 6))
    x0 = x0 + _KS[2]
    x1 = x1 + np.uint32(5)
    return x0 ^ x1


def _score(x_ref, base42, rg, ck, t):
    logits = x_ref[pl.ds(rg * 8, 8), pl.ds(ck * W, W)]
    a = base42 + (t.astype(_U32) * _U32(W) + _U32(np.uint32(rg * 8 * C)))
    bits = _threefry_bits(a)
    m = jax.lax.shift_right_logical(bits, _U32(9))
    f = jax.lax.convert_element_type(m, jnp.float32) * np.float32(2.0 ** -23)
    u = jnp.maximum(_TINY, f)
    nl2 = jnp.log(-jnp.log(u))
    return logits - nl2


def _sample_kernel(x_ref, o_ref, acc_val, acc_t):
    j = pl.program_id(0)

    @pl.when(j == 0)
    def _():
        acc_val[...] = jnp.full((R, W), -jnp.inf, jnp.float32)
        acc_t[...] = jnp.zeros((R, W), jnp.int32)

    lane = jax.lax.broadcasted_iota(jnp.int32, (8, W), 1)
    row_c = jax.lax.broadcasted_iota(jnp.int32, (8, W), 0) * np.int32(C)
    base42 = jax.lax.bitcast_convert_type(lane + row_c, _U32) + _U32(42)

    @pl.when(j != NBLK - 1)
    def _():
        for rg in range(RG):
            av = acc_val[pl.ds(rg * 8, 8), :]
            at = acc_t[pl.ds(rg * 8, 8), :]
            for ck in range(CHUNKS):
                t = j * np.int32(CHUNKS) + np.int32(ck)
                score = _score(x_ref, base42, rg, ck, t)
                take = score > av
                av = jnp.where(take, score, av)
                at = jnp.where(take, t, at)
            acc_val[pl.ds(rg * 8, 8), :] = av
            acc_t[pl.ds(rg * 8, 8), :] = at

    @pl.when(j == NBLK - 1)
    def _():
        for rg in range(RG):
            av = acc_val[pl.ds(rg * 8, 8), :]
            at = acc_t[pl.ds(rg * 8, 8), :]
            for ck in range(TAIL_FULL + (1 if TAIL_REM else 0)):
                t = np.int32((NBLK - 1) * CHUNKS + ck)
                score = _score(x_ref, base42, rg, ck, jnp.int32(t))
                take = score > av
                if ck >= TAIL_FULL:
                    take = take & (lane < np.int32(TAIL_REM))
                av = jnp.where(take, score, av)
                at = jnp.where(take, t, at)
            col = at * np.int32(W) + lane
            rmax = jnp.max(av, axis=1, keepdims=True)
            cand = jnp.where(av == rmax, col, _IMAX)
            o_ref[pl.ds(rg * 8, 8), :] = jnp.min(cand, axis=1, keepdims=True)


def kernel(logits):
    out = pl.pallas_call(
        _sample_kernel,
        grid=(NBLK,),
        in_specs=[pl.BlockSpec((R, C_BLK), lambda j: (0, j))],
        out_specs=pl.BlockSpec((R, 1), lambda j: (0, 0)),
        out_shape=jax.ShapeDtypeStruct((R, 1), jnp.int32),
        scratch_shapes=[
            pltpu.VMEM((R, W), jnp.float32),
            pltpu.VMEM((R, W), jnp.int32),
        ],
        compiler_params=pltpu.CompilerParams(
            dimension_semantics=("arbitrary",),
        ),
    )(logits)
    return out.reshape(R)


# speedup vs baseline: 1.5875x; 1.0036x over previous
import numpy as np
import jax
import jax.numpy as jnp
from jax.experimental import pallas as pl
from jax.experimental.pallas import tpu as pltpu

R = 64
C = 100000
C_BLK = 16384
NBLK = (C + C_BLK - 1) // C_BLK
W = 2048
CHUNKS = C_BLK // W
RG = R // 8
TAIL = C - (NBLK - 1) * C_BLK
TAIL_FULL = TAIL // W
TAIL_REM = TAIL - TAIL_FULL * W

_U32 = jnp.uint32
_KS = (np.uint32(0), np.uint32(42), np.uint32(42 ^ 0x1BD11BDA))
_TINY = np.float32(np.finfo(np.float32).tiny)
_IMAX = np.int32(np.iinfo(np.int32).max)


def _rotl(v, d):
    return (v << _U32(d)) | jax.lax.shift_right_logical(v, _U32(32 - d))


def _threefry_bits(a):
    x0 = a
    x1 = _rotl(a, 13) ^ a

    def rounds(x0, x1, rots):
        for r in rots:
            x0 = x0 + x1
            x1 = _rotl(x1, r) ^ x0
        return x0, x1

    x0, x1 = rounds(x0, x1, (15, 26, 6))
    x0 = x0 + _KS[1]
    x1 = x1 + np.uint32(_KS[2] + np.uint32(1))
    x0, x1 = rounds(x0, x1, (17, 29, 16, 24))
    x0 = x0 + _KS[2]
    x1 = x1 + np.uint32(2)
    x0, x1 = rounds(x0, x1, (13, 15, 26, 6))
    x1 = x1 + np.uint32(_KS[1] + np.uint32(3))
    x0, x1 = rounds(x0, x1, (17, 29, 16, 24))
    x0 = x0 + _KS[1]
    x1 = x1 + np.uint32(_KS[2] + np.uint32(4))
    x0, x1 = rounds(x0, x1, (13, 15, 26, 6))
    x0 = x0 + _KS[2]
    x1 = x1 + np.uint32(5)
    return x0 ^ x1


def _score(x_ref, base42, rg, ck, t):
    logits = x_ref[pl.ds(rg * 8, 8), pl.ds(ck * W, W)]
    a = base42 + (t.astype(_U32) * _U32(W) + _U32(np.uint32(rg * 8 * C)))
    bits = _threefry_bits(a)
    m = jax.lax.shift_right_logical(bits, _U32(9))
    f = jax.lax.convert_element_type(m, jnp.float32) * np.float32(2.0 ** -23)
    u = jnp.maximum(_TINY, f)
    nl2 = jnp.log(-jnp.log(u))
    return logits - nl2


def _sample_kernel(x_ref, o_ref, acc_val, acc_t):
    j = pl.program_id(0)

    @pl.when(j == 0)
    def _():
        acc_val[...] = jnp.full((R, W), -jnp.inf, jnp.float32)
        acc_t[...] = jnp.zeros((R, W), jnp.int32)

    lane = jax.lax.broadcasted_iota(jnp.int32, (8, W), 1)
    row_c = jax.lax.broadcasted_iota(jnp.int32, (8, W), 0) * np.int32(C)
    base42 = jax.lax.bitcast_convert_type(lane + row_c, _U32) + _U32(42)

    @pl.when(j != NBLK - 1)
    def _():
        for rg in range(RG):
            av = acc_val[pl.ds(rg * 8, 8), :]
            at = acc_t[pl.ds(rg * 8, 8), :]
            for ck in range(CHUNKS):
                t = j * np.int32(CHUNKS) + np.int32(ck)
                score = _score(x_ref, base42, rg, ck, t)
                take = score > av
                av = jnp.where(take, score, av)
                at = jnp.where(take, t, at)
            acc_val[pl.ds(rg * 8, 8), :] = av
            acc_t[pl.ds(rg * 8, 8), :] = at

    @pl.when(j == NBLK - 1)
    def _():
        for rg in range(RG):
            av = acc_val[pl.ds(rg * 8, 8), :]
            at = acc_t[pl.ds(rg * 8, 8), :]
            for ck in range(TAIL_FULL + (1 if TAIL_REM else 0)):
                t = np.int32((NBLK - 1) * CHUNKS + ck)
                score = _score(x_ref, base42, rg, ck, jnp.int32(t))
                take = score > av
                if ck >= TAIL_FULL:
                    take = take & (lane < np.int32(TAIL_REM))
                av = jnp.where(take, score, av)
                at = jnp.where(take, t, at)
            col = at * np.int32(W) + lane
            rmax = jnp.max(av, axis=1, keepdims=True)
            cand = jnp.where(av == rmax, col, _IMAX)
            o_ref[pl.ds(rg * 8, 8), :] = jnp.min(cand, axis=1, keepdims=True)


def kernel(logits):
    out = pl.pallas_call(
        _sample_kernel,
        grid=(NBLK,),
        in_specs=[pl.BlockSpec((R, C_BLK), lambda j: (0, j))],
        out_specs=pl.BlockSpec((R, 1), lambda j: (0, 0)),
        out_shape=jax.ShapeDtypeStruct((R, 1), jnp.int32),
        scratch_shapes=[
            pltpu.VMEM((R, W), jnp.float32),
            pltpu.VMEM((R, W), jnp.int32),
        ],
        compiler_params=pltpu.CompilerParams(
            dimension_semantics=("arbitrary",),
        ),
    )(logits)
    return out.reshape(R)


# speedup vs baseline: 1.5906x; 1.0019x over previous
import numpy as np
import jax
import jax.numpy as jnp
from jax.experimental import pallas as pl
from jax.experimental.pallas import tpu as pltpu

R = 64
C = 100000
C_BLK = 16384
NBLK = (C + C_BLK - 1) // C_BLK
W = 1024
CHUNKS = C_BLK // W
RG = R // 8
TAIL = C - (NBLK - 1) * C_BLK
TAIL_FULL = TAIL // W
TAIL_REM = TAIL - TAIL_FULL * W

_U32 = jnp.uint32
_KS = (np.uint32(0), np.uint32(42), np.uint32(42 ^ 0x1BD11BDA))
_TINY = np.float32(np.finfo(np.float32).tiny)
_IMAX = np.int32(np.iinfo(np.int32).max)


def _rotl(v, d):
    return (v << _U32(d)) | jax.lax.shift_right_logical(v, _U32(32 - d))


def _threefry_bits(a):
    x0 = a
    x1 = _rotl(a, 13) ^ a

    def rounds(x0, x1, rots):
        for r in rots:
            x0 = x0 + x1
            x1 = _rotl(x1, r) ^ x0
        return x0, x1

    x0, x1 = rounds(x0, x1, (15, 26, 6))
    x0 = x0 + _KS[1]
    x1 = x1 + np.uint32(_KS[2] + np.uint32(1))
    x0, x1 = rounds(x0, x1, (17, 29, 16, 24))
    x0 = x0 + _KS[2]
    x1 = x1 + np.uint32(2)
    x0, x1 = rounds(x0, x1, (13, 15, 26, 6))
    x1 = x1 + np.uint32(_KS[1] + np.uint32(3))
    x0, x1 = rounds(x0, x1, (17, 29, 16, 24))
    x0 = x0 + _KS[1]
    x1 = x1 + np.uint32(_KS[2] + np.uint32(4))
    x0, x1 = rounds(x0, x1, (13, 15, 26, 6))
    x0 = x0 + _KS[2]
    x1 = x1 + np.uint32(5)
    return x0 ^ x1


def _score(x_ref, base42, rg, ck, t):
    logits = x_ref[pl.ds(rg * 8, 8), pl.ds(ck * W, W)]
    a = base42 + (t.astype(_U32) * _U32(W) + _U32(np.uint32(rg * 8 * C)))
    bits = _threefry_bits(a)
    m = jax.lax.shift_right_logical(bits, _U32(9))
    f = jax.lax.convert_element_type(m, jnp.float32) * np.float32(2.0 ** -23)
    u = jnp.maximum(_TINY, f)
    nl2 = jnp.log(-jnp.log(u))
    return logits - nl2


def _sample_kernel(x_ref, o_ref, acc_val, acc_t):
    j = pl.program_id(0)

    @pl.when(j == 0)
    def _():
        acc_val[...] = jnp.full((R, W), -jnp.inf, jnp.float32)
        acc_t[...] = jnp.zeros((R, W), jnp.int32)

    lane = jax.lax.broadcasted_iota(jnp.int32, (8, W), 1)
    row_c = jax.lax.broadcasted_iota(jnp.int32, (8, W), 0) * np.int32(C)
    base42 = jax.lax.bitcast_convert_type(lane + row_c, _U32) + _U32(42)

    @pl.when(j != NBLK - 1)
    def _():
        for rg in range(RG):
            av = acc_val[pl.ds(rg * 8, 8), :]
            at = acc_t[pl.ds(rg * 8, 8), :]
            for ck in range(CHUNKS):
                t = j * np.int32(CHUNKS) + np.int32(ck)
                score = _score(x_ref, base42, rg, ck, t)
                take = score > av
                av = jnp.where(take, score, av)
                at = jnp.where(take, t, at)
            acc_val[pl.ds(rg * 8, 8), :] = av
            acc_t[pl.ds(rg * 8, 8), :] = at

    @pl.when(j == NBLK - 1)
    def _():
        for rg in range(RG):
            av = acc_val[pl.ds(rg * 8, 8), :]
            at = acc_t[pl.ds(rg * 8, 8), :]
            for ck in range(TAIL_FULL + (1 if TAIL_REM else 0)):
                t = np.int32((NBLK - 1) * CHUNKS + ck)
                score = _score(x_ref, base42, rg, ck, jnp.int32(t))
                take = score > av
                if ck >= TAIL_FULL:
                    take = take & (lane < np.int32(TAIL_REM))
                av = jnp.where(take, score, av)
                at = jnp.where(take, t, at)
            col = at * np.int32(W) + lane
            rmax = jnp.max(av, axis=1, keepdims=True)
            cand = jnp.where(av == rmax, col, _IMAX)
            o_ref[pl.ds(rg * 8, 8), :] = jnp.min(cand, axis=1, keepdims=True)


def kernel(logits):
    out = pl.pallas_call(
        _sample_kernel,
        grid=(NBLK,),
        in_specs=[pl.BlockSpec((R, C_BLK), lambda j: (0, j))],
        out_specs=pl.BlockSpec((R, 1), lambda j: (0, 0)),
        out_shape=jax.ShapeDtypeStruct((R, 1), jnp.int32),
        scratch_shapes=[
            pltpu.VMEM((R, W), jnp.float32),
            pltpu.VMEM((R, W), jnp.int32),
        ],
        compiler_params=pltpu.CompilerParams(
            dimension_semantics=("arbitrary",),
        ),
    )(logits)
    return out.reshape(R)
